# baseline (speedup 1.0000x reference)
import functools

import jax
import jax.numpy as jnp
from jax import lax
from jax.experimental import pallas as pl
from jax.experimental.pallas import tpu as pltpu
from jax.experimental.pallas import tpu_sc as plsc

DIM = 64
B = 16384
CTX = 20
NEG = 20
NSLOT = 1 + NEG
NSLOT_PAD = 32

NC = 2
NS = 16
NW = NC * NS
BW = B // NW
C = 16
G = BW // C
PER_W = BW * NSLOT_PAD
TOT = B * NSLOT_PAD
LG = 16
NJ = DIM // LG

VOCAB = 1000000
PACK_BLK = 8192
PACK_NBLK = (VOCAB + PACK_BLK - 1) // PACK_BLK
PACK_ROWS = PACK_NBLK * (PACK_BLK // 2)
PACK_SH = PACK_BLK.bit_length() - 1
PACK_HMASK = PACK_BLK // 2 - 1


def _sc_body(xp_hbm, xn_hbm, y_hbm, v_hbm, u_hbm, out_hbm,
             idxp_v, idxn_v, idxy_v,
             vr0, vr1, uy0, uy1, un0, un1, out_v, sem0, sem1):
    wid = lax.axis_index("s") * NC + lax.axis_index("c")
    wbase = wid * BW

    pltpu.sync_copy(xp_hbm.at[pl.ds(wbase * CTX, BW * CTX)], idxp_v)
    pltpu.sync_copy(xn_hbm.at[pl.ds(wbase * NEG, BW * NEG)], idxn_v)
    pltpu.sync_copy(y_hbm.at[pl.ds(wbase, BW)], idxy_v)

    def remap(ref, n):
        def rbody(k, carry):
            r = ref[pl.ds(k * LG, LG)]
            q = (((r >> PACK_SH) << PACK_SH)
                 + ((r & PACK_HMASK) << 1) + ((r >> (PACK_SH - 1)) & 1))
            ref[pl.ds(k * LG, LG)] = q
            return carry

        lax.fori_loop(0, n // LG, rbody, 0)

    remap(idxp_v, BW * CTX)
    remap(idxn_v, BW * NEG)
    remap(idxy_v, BW)

    def fire(g, vr, uy, un, sem):
        p0 = g * (C * CTX)
        pltpu.async_copy(v_hbm.at[idxp_v.at[pl.ds(p0, 128)]],
                         vr.at[pl.ds(0, 128)], sem)
        pltpu.async_copy(v_hbm.at[idxp_v.at[pl.ds(p0 + 128, 128)]],
                         vr.at[pl.ds(128, 128)], sem)
        pltpu.async_copy(v_hbm.at[idxp_v.at[pl.ds(p0 + 256, 64)]],
                         vr.at[pl.ds(256, 64)], sem)
        pltpu.async_copy(u_hbm.at[idxy_v.at[pl.ds(g * C, C)]], uy, sem)
        n0 = g * (C * NEG)
        pltpu.async_copy(u_hbm.at[idxn_v.at[pl.ds(n0, 128)]],
                         un.at[pl.ds(0, 128)], sem)
        pltpu.async_copy(u_hbm.at[idxn_v.at[pl.ds(n0 + 128, 128)]],
                         un.at[pl.ds(128, 128)], sem)
        pltpu.async_copy(u_hbm.at[idxn_v.at[pl.ds(n0 + 256, 64)]],
                         un.at[pl.ds(256, 64)], sem)

    def drain(vr, uy, un, sem):
        pltpu.make_async_copy(v_hbm.at[pl.ds(0, 128)], vr.at[pl.ds(0, 128)], sem).wait()
        pltpu.make_async_copy(v_hbm.at[pl.ds(0, 128)], vr.at[pl.ds(128, 128)], sem).wait()
        pltpu.make_async_copy(v_hbm.at[pl.ds(0, 64)], vr.at[pl.ds(256, 64)], sem).wait()
        pltpu.make_async_copy(u_hbm.at[pl.ds(0, C)], uy, sem).wait()
        pltpu.make_async_copy(u_hbm.at[pl.ds(0, 128)], un.at[pl.ds(0, 128)], sem).wait()
        pltpu.make_async_copy(u_hbm.at[pl.ds(0, 128)], un.at[pl.ds(128, 128)], sem).wait()
        pltpu.make_async_copy(u_hbm.at[pl.ds(0, 64)], un.at[pl.ds(256, 64)], sem).wait()

    lane = lax.iota(jnp.int32, LG)
    perms = {d: lane ^ d for d in (1, 2, 4, 8)}
    masks = {d: (lane & d) != 0 for d in (1, 2, 4, 8)}

    dnums = lax.GatherDimensionNumbers(
        offset_dims=(), collapsed_slice_dims=(0,), start_index_map=(0,))

    def shuffle(u, perm):
        return lax.gather(u, perm[:, None], dimension_numbers=dnums,
                          slice_sizes=(1,),
                          mode=lax.GatherScatterMode.PROMISE_IN_BOUNDS)

    def merge(a, b, d):
        t = jnp.where(masks[d], b, a)
        u = jnp.where(masks[d], a, b)
        return t + shuffle(u, perms[d])

    def tree(vs):
        d = 1
        while len(vs) > 1:
            vs = [merge(vs[k], vs[k + 1], d) for k in range(0, len(vs), 2)]
            d *= 2
        return vs[0]

    def compute(g, vr, uy, un):
        def body(i, carry):
            row0 = i * CTX
            hs = []
            for j in range(NJ):
                acc = vr[row0, pl.ds(j * LG, LG)]
                for r in range(1, CTX):
                    acc = acc + vr[row0 + r, pl.ds(j * LG, LG)]
                hs.append(acc * (1.0 / CTX))
            accs = []
            accd = hs[0] * uy[i, pl.ds(0, LG)]
            for j in range(1, NJ):
                accd = accd + hs[j] * uy[i, pl.ds(j * LG, LG)]
            accs.append(accd)
            for n in range(NEG):
                rr = row0 + n
                accn = hs[0] * un[rr, pl.ds(0, LG)]
                for j in range(1, NJ):
                    accn = accn + hs[j] * un[rr, pl.ds(j * LG, LG)]
                accs.append(-accn)
            zero = jnp.zeros((LG,), jnp.float32)
            slo = tree(accs[:LG])
            shi = tree(accs[LG:] + [zero] * (2 * LG - NSLOT))
            obase = (g * C + i) * NSLOT_PAD
            out_v[pl.ds(obase, LG)] = slo
            out_v[pl.ds(obase + LG, LG)] = shi
            return carry

        lax.fori_loop(0, C, body, 0)

    fire(0, vr0, uy0, un0, sem0)
    fire(1, vr1, uy1, un1, sem1)

    def outer(gg, carry):
        g0 = gg * 2
        drain(vr0, uy0, un0, sem0)
        compute(g0, vr0, uy0, un0)

        @pl.when(g0 + 2 < G)
        def _():
            fire(g0 + 2, vr0, uy0, un0, sem0)

        g1 = g0 + 1
        drain(vr1, uy1, un1, sem1)
        compute(g1, vr1, uy1, un1)

        @pl.when(g1 + 2 < G)
        def _():
            fire(g1 + 2, vr1, uy1, un1, sem1)

        return carry

    lax.fori_loop(0, G // 2, outer, 0)

    pltpu.sync_copy(out_v, out_hbm.at[pl.ds(wid * PER_W, PER_W)])


def _make_sc_dots():
    mesh = plsc.VectorSubcoreMesh(core_axis_name="c", subcore_axis_name="s")
    return pl.kernel(
        _sc_body,
        mesh=mesh,
        compiler_params=pltpu.CompilerParams(use_tc_tiling_on_sc=False),
        out_type=jax.ShapeDtypeStruct((TOT,), jnp.float32),
        scratch_types=[
            pltpu.VMEM((BW * CTX,), jnp.int32),
            pltpu.VMEM((BW * NEG,), jnp.int32),
            pltpu.VMEM((BW,), jnp.int32),
            pltpu.VMEM((C * CTX, DIM), jnp.float32),
            pltpu.VMEM((C * CTX, DIM), jnp.float32),
            pltpu.VMEM((C, DIM), jnp.float32),
            pltpu.VMEM((C, DIM), jnp.float32),
            pltpu.VMEM((C * NEG, DIM), jnp.float32),
            pltpu.VMEM((C * NEG, DIM), jnp.float32),
            pltpu.VMEM((PER_W,), jnp.float32),
            pltpu.SemaphoreType.DMA,
            pltpu.SemaphoreType.DMA,
        ],
    )


def _pack_body(vt_ref, out_ref):
    x = vt_ref[...]
    gcol = (pl.program_id(0) * PACK_BLK
            + lax.broadcasted_iota(jnp.int32, x.shape, 1))
    x = jnp.where(gcol < VOCAB, x, 0.0)
    hi = x.astype(jnp.bfloat16)
    lo = (x - hi.astype(jnp.float32)).astype(jnp.bfloat16)
    r = lax.broadcasted_iota(jnp.int32, (DIM, 128), 0)
    c = lax.broadcasted_iota(jnp.int32, (DIM, 128), 1)
    e0 = (r == c).astype(jnp.bfloat16)
    e1 = (r == (c - DIM)).astype(jnp.bfloat16)
    dn = (((0,), (0,)), ((), ()))
    HB = PACK_BLK // 2

    def tdot(a, e):
        return lax.dot_general(a, e, dn, preferred_element_type=jnp.float32)

    out_ref[...] = (tdot(hi[:, :HB], e0) + tdot(lo[:, :HB], e0)
                    + tdot(hi[:, HB:], e1) + tdot(lo[:, HB:], e1))


def _pack(vt):
    return pl.pallas_call(
        _pack_body,
        grid=(PACK_NBLK,),
        in_specs=[pl.BlockSpec((64, PACK_BLK), lambda i: (0, i))],
        out_specs=pl.BlockSpec((PACK_BLK // 2, 128), lambda i: (i, 0)),
        out_shape=jax.ShapeDtypeStruct((PACK_ROWS, 128), jnp.float32),
    )(vt)


def _finish_body(dots_ref, out_ref):
    x = dots_ref[...]
    col = lax.broadcasted_iota(jnp.int32, x.shape, 1)
    real = (col % NSLOT_PAD) < NSLOT
    ls = jnp.minimum(x, 0.0) - jnp.log1p(jnp.exp(-jnp.abs(x)))
    out_ref[0, 0] = -jnp.sum(jnp.where(real, ls, 0.0)) * (1.0 / B)


def _finish(dots2d):
    return pl.pallas_call(
        _finish_body,
        out_shape=jax.ShapeDtypeStruct((1, 1), jnp.float32),
        out_specs=pl.BlockSpec(memory_space=pltpu.SMEM),
    )(dots2d)


def kernel(x_positive, x_negative, y, embedding_v, embedding_u):
    vp = _pack(embedding_v.T).reshape(2 * PACK_ROWS, DIM)
    up = _pack(embedding_u.T).reshape(2 * PACK_ROWS, DIM)
    return vp[0, 0] + up[0, 0]
    sc_dots = _make_sc_dots()
    dots = sc_dots(x_positive.reshape(-1), x_negative.reshape(-1), y, vp, up)
    res = _finish(dots.reshape(TOT // 128, 128))
    return res[0, 0]


# speedup vs baseline: 2.0407x; 2.0407x over previous
import functools

import jax
import jax.numpy as jnp
from jax import lax
from jax.experimental import pallas as pl
from jax.experimental.pallas import tpu as pltpu
from jax.experimental.pallas import tpu_sc as plsc

DIM = 64
B = 16384
CTX = 20
NEG = 20
NSLOT = 1 + NEG
NSLOT_PAD = 32

NC = 2
NS = 16
NW = NC * NS
BW = B // NW
C = 16
G = BW // C
PER_W = BW * NSLOT_PAD
TOT = B * NSLOT_PAD
LG = 16
NJ = DIM // LG

VOCAB = 1000000
PACK_BLK = 8192
PACK_NBLK = (VOCAB + PACK_BLK - 1) // PACK_BLK
PACK_ROWS = PACK_NBLK * (PACK_BLK // 2)
PACK_SH = PACK_BLK.bit_length() - 1
PACK_HMASK = PACK_BLK // 2 - 1


def _sc_body(xp_hbm, xn_hbm, y_hbm, v_hbm, u_hbm, out_hbm,
             idxp_v, idxn_v, idxy_v,
             vr0, vr1, uy0, uy1, un0, un1, out_v, sem0, sem1):
    wid = lax.axis_index("s") * NC + lax.axis_index("c")
    wbase = wid * BW

    pltpu.sync_copy(xp_hbm.at[pl.ds(wbase * CTX, BW * CTX)], idxp_v)
    pltpu.sync_copy(xn_hbm.at[pl.ds(wbase * NEG, BW * NEG)], idxn_v)
    pltpu.sync_copy(y_hbm.at[pl.ds(wbase, BW)], idxy_v)

    def remap(ref, n):
        def rbody(k, carry):
            r = ref[pl.ds(k * LG, LG)]
            q = (((r >> PACK_SH) << PACK_SH)
                 + ((r & PACK_HMASK) << 1) + ((r >> (PACK_SH - 1)) & 1))
            ref[pl.ds(k * LG, LG)] = q
            return carry

        lax.fori_loop(0, n // LG, rbody, 0)

    remap(idxp_v, BW * CTX)
    remap(idxn_v, BW * NEG)
    remap(idxy_v, BW)

    def fire(g, vr, uy, un, sem):
        p0 = g * (C * CTX)
        pltpu.async_copy(v_hbm.at[idxp_v.at[pl.ds(p0, 128)]],
                         vr.at[pl.ds(0, 128)], sem)
        pltpu.async_copy(v_hbm.at[idxp_v.at[pl.ds(p0 + 128, 128)]],
                         vr.at[pl.ds(128, 128)], sem)
        pltpu.async_copy(v_hbm.at[idxp_v.at[pl.ds(p0 + 256, 64)]],
                         vr.at[pl.ds(256, 64)], sem)
        pltpu.async_copy(u_hbm.at[idxy_v.at[pl.ds(g * C, C)]], uy, sem)
        n0 = g * (C * NEG)
        pltpu.async_copy(u_hbm.at[idxn_v.at[pl.ds(n0, 128)]],
                         un.at[pl.ds(0, 128)], sem)
        pltpu.async_copy(u_hbm.at[idxn_v.at[pl.ds(n0 + 128, 128)]],
                         un.at[pl.ds(128, 128)], sem)
        pltpu.async_copy(u_hbm.at[idxn_v.at[pl.ds(n0 + 256, 64)]],
                         un.at[pl.ds(256, 64)], sem)

    def drain(vr, uy, un, sem):
        pltpu.make_async_copy(v_hbm.at[pl.ds(0, 128)], vr.at[pl.ds(0, 128)], sem).wait()
        pltpu.make_async_copy(v_hbm.at[pl.ds(0, 128)], vr.at[pl.ds(128, 128)], sem).wait()
        pltpu.make_async_copy(v_hbm.at[pl.ds(0, 64)], vr.at[pl.ds(256, 64)], sem).wait()
        pltpu.make_async_copy(u_hbm.at[pl.ds(0, C)], uy, sem).wait()
        pltpu.make_async_copy(u_hbm.at[pl.ds(0, 128)], un.at[pl.ds(0, 128)], sem).wait()
        pltpu.make_async_copy(u_hbm.at[pl.ds(0, 128)], un.at[pl.ds(128, 128)], sem).wait()
        pltpu.make_async_copy(u_hbm.at[pl.ds(0, 64)], un.at[pl.ds(256, 64)], sem).wait()

    lane = lax.iota(jnp.int32, LG)
    perms = {d: lane ^ d for d in (1, 2, 4, 8)}
    masks = {d: (lane & d) != 0 for d in (1, 2, 4, 8)}

    dnums = lax.GatherDimensionNumbers(
        offset_dims=(), collapsed_slice_dims=(0,), start_index_map=(0,))

    def shuffle(u, perm):
        return lax.gather(u, perm[:, None], dimension_numbers=dnums,
                          slice_sizes=(1,),
                          mode=lax.GatherScatterMode.PROMISE_IN_BOUNDS)

    def merge(a, b, d):
        t = jnp.where(masks[d], b, a)
        u = jnp.where(masks[d], a, b)
        return t + shuffle(u, perms[d])

    def tree(vs):
        d = 1
        while len(vs) > 1:
            vs = [merge(vs[k], vs[k + 1], d) for k in range(0, len(vs), 2)]
            d *= 2
        return vs[0]

    def compute(g, vr, uy, un):
        def body(i, carry):
            row0 = i * CTX
            hs = []
            for j in range(NJ):
                acc = vr[row0, pl.ds(j * LG, LG)]
                for r in range(1, CTX):
                    acc = acc + vr[row0 + r, pl.ds(j * LG, LG)]
                hs.append(acc * (1.0 / CTX))
            accs = []
            accd = hs[0] * uy[i, pl.ds(0, LG)]
            for j in range(1, NJ):
                accd = accd + hs[j] * uy[i, pl.ds(j * LG, LG)]
            accs.append(accd)
            for n in range(NEG):
                rr = row0 + n
                accn = hs[0] * un[rr, pl.ds(0, LG)]
                for j in range(1, NJ):
                    accn = accn + hs[j] * un[rr, pl.ds(j * LG, LG)]
                accs.append(-accn)
            zero = jnp.zeros((LG,), jnp.float32)
            slo = tree(accs[:LG])
            shi = tree(accs[LG:] + [zero] * (2 * LG - NSLOT))
            obase = (g * C + i) * NSLOT_PAD
            out_v[pl.ds(obase, LG)] = slo
            out_v[pl.ds(obase + LG, LG)] = shi
            return carry

        lax.fori_loop(0, C, body, 0)

    fire(0, vr0, uy0, un0, sem0)
    fire(1, vr1, uy1, un1, sem1)

    def outer(gg, carry):
        g0 = gg * 2
        drain(vr0, uy0, un0, sem0)
        compute(g0, vr0, uy0, un0)

        @pl.when(g0 + 2 < G)
        def _():
            fire(g0 + 2, vr0, uy0, un0, sem0)

        g1 = g0 + 1
        drain(vr1, uy1, un1, sem1)
        compute(g1, vr1, uy1, un1)

        @pl.when(g1 + 2 < G)
        def _():
            fire(g1 + 2, vr1, uy1, un1, sem1)

        return carry

    lax.fori_loop(0, G // 2, outer, 0)

    pltpu.sync_copy(out_v, out_hbm.at[pl.ds(wid * PER_W, PER_W)])


def _make_sc_dots():
    mesh = plsc.VectorSubcoreMesh(core_axis_name="c", subcore_axis_name="s")
    return pl.kernel(
        _sc_body,
        mesh=mesh,
        compiler_params=pltpu.CompilerParams(use_tc_tiling_on_sc=False),
        out_type=jax.ShapeDtypeStruct((TOT,), jnp.float32),
        scratch_types=[
            pltpu.VMEM((BW * CTX,), jnp.int32),
            pltpu.VMEM((BW * NEG,), jnp.int32),
            pltpu.VMEM((BW,), jnp.int32),
            pltpu.VMEM((C * CTX, DIM), jnp.float32),
            pltpu.VMEM((C * CTX, DIM), jnp.float32),
            pltpu.VMEM((C, DIM), jnp.float32),
            pltpu.VMEM((C, DIM), jnp.float32),
            pltpu.VMEM((C * NEG, DIM), jnp.float32),
            pltpu.VMEM((C * NEG, DIM), jnp.float32),
            pltpu.VMEM((PER_W,), jnp.float32),
            pltpu.SemaphoreType.DMA,
            pltpu.SemaphoreType.DMA,
        ],
    )


def _pack_body(vt_ref, out_ref):
    x = vt_ref[...]
    gcol = (pl.program_id(0) * PACK_BLK
            + lax.broadcasted_iota(jnp.int32, x.shape, 1))
    x = jnp.where(gcol < VOCAB, x, 0.0)
    hi = x.astype(jnp.bfloat16)
    lo = (x - hi.astype(jnp.float32)).astype(jnp.bfloat16)
    r = lax.broadcasted_iota(jnp.int32, (DIM, 128), 0)
    c = lax.broadcasted_iota(jnp.int32, (DIM, 128), 1)
    e0 = (r == c).astype(jnp.bfloat16)
    e1 = (r == (c - DIM)).astype(jnp.bfloat16)
    dn = (((0,), (0,)), ((), ()))
    HB = PACK_BLK // 2

    def tdot(a, e):
        return lax.dot_general(a, e, dn, preferred_element_type=jnp.float32)

    out_ref[...] = (tdot(hi[:, :HB], e0) + tdot(lo[:, :HB], e0)
                    + tdot(hi[:, HB:], e1) + tdot(lo[:, HB:], e1))


def _pack(vt):
    return pl.pallas_call(
        _pack_body,
        grid=(PACK_NBLK,),
        in_specs=[pl.BlockSpec((64, PACK_BLK), lambda i: (0, i))],
        out_specs=pl.BlockSpec((PACK_BLK // 2, 128), lambda i: (i, 0)),
        out_shape=jax.ShapeDtypeStruct((PACK_ROWS, 128), jnp.float32),
    )(vt)


def _finish_body(dots_ref, out_ref):
    x = dots_ref[...]
    col = lax.broadcasted_iota(jnp.int32, x.shape, 1)
    real = (col % NSLOT_PAD) < NSLOT
    ls = jnp.minimum(x, 0.0) - jnp.log1p(jnp.exp(-jnp.abs(x)))
    out_ref[0, 0] = -jnp.sum(jnp.where(real, ls, 0.0)) * (1.0 / B)


def _finish(dots2d):
    return pl.pallas_call(
        _finish_body,
        out_shape=jax.ShapeDtypeStruct((1, 1), jnp.float32),
        out_specs=pl.BlockSpec(memory_space=pltpu.SMEM),
    )(dots2d)


def kernel(x_positive, x_negative, y, embedding_v, embedding_u):
    vp = _pack(embedding_v.T).reshape(2 * PACK_ROWS, DIM)
    up = _pack(embedding_u.T).reshape(2 * PACK_ROWS, DIM)
    sc_dots = _make_sc_dots()
    dots = sc_dots(x_positive.reshape(-1), x_negative.reshape(-1), y, vp, up)
    res = _finish(dots.reshape(TOT // 128, 128))
    return res[0, 0]
